# baseline (speedup 1.0000x reference)
import functools

import jax
import jax.numpy as jnp
from jax import lax
from jax.experimental import pallas as pl
from jax.experimental.pallas import tpu as pltpu
from jax.experimental.pallas import tpu_sc as plsc

N_NODES = 100000
HID = 128
NUM_CLASS = 32
NUM_SEG = 256

_C = 128
_NCH = N_NODES // _C
_TAIL = N_NODES - _NCH * _C
_NC = 2
_NS = 16
_NW = _NC * _NS
_CW = 16


def _sc_body(x_hbm, ids_hbm, ones_hbm, zc_hbm, sums_hbm, cnt_hbm,
             xbuf, idbuf, ones_v, zer_v, zc_v, acc_sh, cnt_sh,
             sem_x, sem_i, sem_s, sem_c):
    cid = lax.axis_index("c")
    sid = lax.axis_index("s")
    wid = sid * _NC + cid

    z16 = jnp.zeros((16,), jnp.float32)
    for r in range(16):
        for j in range(HID // 16):
            zer_v[r, pl.ds(j * 16, 16)] = z16
    pltpu.sync_copy(ones_hbm, ones_v)
    pltpu.sync_copy(zer_v, acc_sh.at[pl.ds(sid * 16, 16)])

    @pl.when(sid == 0)
    def _zero_counts():
        pltpu.sync_copy(zc_hbm, zc_v)
        pltpu.sync_copy(zc_v, cnt_sh)

    plsc.subcore_barrier()

    nchunks = (_NCH - wid + _NW - 1) // _NW

    def chunk_step(i, carry):
        base = (wid + i * _NW) * _C
        pltpu.async_copy(ids_hbm.at[pl.ds(base, _C)], idbuf.at[0], sem_i).wait()
        pltpu.async_copy(x_hbm.at[pl.ds(base, _C), :], xbuf, sem_x).wait()
        cp_s = pltpu.async_copy(xbuf, acc_sh.at[idbuf.at[0]], sem_s, add=True)
        pltpu.async_copy(ones_v, cnt_sh.at[idbuf.at[0]], sem_c, add=True).wait()
        cp_s.wait()
        return carry

    lax.fori_loop(0, nchunks, chunk_step, 0)

    plsc.subcore_barrier()
    pltpu.sync_copy(acc_sh.at[pl.ds(sid * 16, 16)],
                    sums_hbm.at[cid, pl.ds(sid * 16, 16)])

    @pl.when(sid == 0)
    def _pub_counts():
        pltpu.sync_copy(cnt_sh, cnt_hbm.at[cid])


_sc_segsum = functools.partial(
    pl.kernel,
    out_type=(jax.ShapeDtypeStruct((_NC, NUM_SEG, HID), jnp.float32),
              jax.ShapeDtypeStruct((_NC, NUM_SEG), jnp.float32)),
    mesh=plsc.VectorSubcoreMesh(core_axis_name="c", subcore_axis_name="s",
                                num_cores=_NC, num_subcores=_NS),
    scratch_types=[
        pltpu.VMEM((_C, HID), jnp.float32),
        pltpu.VMEM((1, _C), jnp.int32),
        pltpu.VMEM((_C,), jnp.float32),
        pltpu.VMEM((16, HID), jnp.float32),
        pltpu.VMEM((NUM_SEG,), jnp.float32),
        pltpu.VMEM_SHARED((NUM_SEG, HID), jnp.float32),
        pltpu.VMEM_SHARED((NUM_SEG,), jnp.float32),
        pltpu.SemaphoreType.DMA,
        pltpu.SemaphoreType.DMA,
        pltpu.SemaphoreType.DMA,
        pltpu.SemaphoreType.DMA,
    ],
)(_sc_body)


def _tc_body(sums_ref, cnt_ref, xt_ref, idt_ref, emb_ref, W_ref, b_ref, out_ref):
    sums = sums_ref[0] + sums_ref[1]
    seg_iota = jax.lax.broadcasted_iota(jnp.int32, (NUM_SEG, _TAIL), 0)
    onehot_t = (idt_ref[...] == seg_iota.astype(jnp.float32)).astype(jnp.float32)
    sums = sums + jax.lax.dot(onehot_t, xt_ref[...],
                              preferred_element_type=jnp.float32)
    counts = cnt_ref[0] + cnt_ref[1] + jnp.sum(onehot_t, axis=1)
    mean = sums / jnp.maximum(counts, 1.0)[:, None]
    cat = jnp.concatenate([mean, emb_ref[...]], axis=1)
    out_ref[...] = jax.lax.dot_general(
        cat, W_ref[...], (((1,), (1,)), ((), ())),
        preferred_element_type=jnp.float32) + b_ref[...]


def kernel(x, segment_ids, emb, W, b):
    ids = segment_ids.astype(jnp.int32)
    ones_c = jnp.ones((_C,), jnp.float32)
    zc_c = jnp.zeros((NUM_SEG,), jnp.float32)
    sums_p, cnt_p = _sc_segsum(x, ids, ones_c, zc_c)
    x_tail = jax.lax.slice(x, (_NCH * _C, 0), (N_NODES, HID))
    ids_tail = ids[_NCH * _C:].astype(jnp.float32).reshape(_TAIL, 1)
    out = pl.pallas_call(
        _tc_body,
        out_shape=jax.ShapeDtypeStruct((NUM_SEG, NUM_CLASS), jnp.float32),
    )(sums_p, cnt_p, x_tail, jnp.transpose(ids_tail), emb, W, b.reshape(1, NUM_CLASS))
    return out


# speedup vs baseline: 1.3907x; 1.3907x over previous
import functools

import jax
import jax.numpy as jnp
from jax import lax
from jax.experimental import pallas as pl
from jax.experimental.pallas import tpu as pltpu
from jax.experimental.pallas import tpu_sc as plsc

N_NODES = 100000
HID = 128
NUM_CLASS = 32
NUM_SEG = 256

_C = 128
_NCH = N_NODES // _C
_TAIL = N_NODES - _NCH * _C
_NC = 2
_NS = 16
_NW = _NC * _NS
_MAXCH = (_NCH + _NW - 1) // _NW
_NLONG = _NCH - (_MAXCH - 1) * _NW
_IDPAD = _NW * _MAXCH


def _sc_body(x_hbm, ids2_hbm, ones_hbm, zc_hbm, sums_hbm, cnt_hbm,
             xbuf0, xbuf1, idbuf, ones_v, zer_v, zc_v, acc_sh, cnt_sh,
             sem_x, sem_i, sem_s, sem_c):
    cid = lax.axis_index("c")
    sid = lax.axis_index("s")
    wid = sid * _NC + cid

    extra = jnp.minimum(wid, _NLONG)
    c0 = wid * (_MAXCH - 1) + extra
    n = jnp.where(wid < _NLONG, _MAXCH, _MAXCH - 1)

    cp_ids = pltpu.async_copy(ids2_hbm.at[pl.ds(c0, _MAXCH)], idbuf, sem_i)
    pltpu.sync_copy(ones_hbm, ones_v)

    z16 = jnp.zeros((16,), jnp.float32)
    for r in range(16):
        for j in range(HID // 16):
            zer_v[r, pl.ds(j * 16, 16)] = z16
    pltpu.sync_copy(zer_v, acc_sh.at[pl.ds(sid * 16, 16)])

    @pl.when(sid == 0)
    def _zero_counts():
        pltpu.sync_copy(zc_hbm, zc_v)
        pltpu.sync_copy(zc_v, cnt_sh)

    cp_ids.wait()
    plsc.subcore_barrier()

    def x_chunk(i):
        return x_hbm.at[pl.ds((c0 + i) * _C, _C), :]

    def issue_load(i, buf):
        pltpu.async_copy(x_chunk(i), buf, sem_x)

    def wait_load(i, buf):
        pltpu.make_async_copy(x_chunk(i), buf, sem_x).wait()

    def scatter(i, buf):
        cp_s = pltpu.async_copy(buf, acc_sh.at[idbuf.at[i, 0]], sem_s, add=True)
        pltpu.async_copy(ones_v, cnt_sh.at[idbuf.at[i, 0]], sem_c, add=True).wait()
        cp_s.wait()

    issue_load(0, xbuf0)
    npairs = (_MAXCH + 1) // 2

    def pair_step(p, carry):
        i0 = 2 * p
        i1 = i0 + 1

        @pl.when(i0 < n)
        def _even():
            wait_load(i0, xbuf0)

            @pl.when(i1 < n)
            def _pf1():
                issue_load(i1, xbuf1)

            scatter(i0, xbuf0)

            @pl.when(i1 < n)
            def _odd():
                wait_load(i1, xbuf1)

                @pl.when(i1 + 1 < n)
                def _pf0():
                    issue_load(i1 + 1, xbuf0)

                scatter(i1, xbuf1)

        return carry

    lax.fori_loop(0, npairs, pair_step, 0)

    plsc.subcore_barrier()
    pltpu.sync_copy(acc_sh.at[pl.ds(sid * 16, 16)],
                    sums_hbm.at[cid, pl.ds(sid * 16, 16)])

    @pl.when(sid == 0)
    def _pub_counts():
        pltpu.sync_copy(cnt_sh, cnt_hbm.at[cid])


_sc_segsum = functools.partial(
    pl.kernel,
    out_type=(jax.ShapeDtypeStruct((_NC, NUM_SEG, HID), jnp.float32),
              jax.ShapeDtypeStruct((_NC, NUM_SEG), jnp.float32)),
    mesh=plsc.VectorSubcoreMesh(core_axis_name="c", subcore_axis_name="s",
                                num_cores=_NC, num_subcores=_NS),
    scratch_types=[
        pltpu.VMEM((_C, HID), jnp.float32),
        pltpu.VMEM((_C, HID), jnp.float32),
        pltpu.VMEM((_MAXCH, 1, _C), jnp.int32),
        pltpu.VMEM((_C,), jnp.float32),
        pltpu.VMEM((16, HID), jnp.float32),
        pltpu.VMEM((NUM_SEG,), jnp.float32),
        pltpu.VMEM_SHARED((NUM_SEG, HID), jnp.float32),
        pltpu.VMEM_SHARED((NUM_SEG,), jnp.float32),
        pltpu.SemaphoreType.DMA,
        pltpu.SemaphoreType.DMA,
        pltpu.SemaphoreType.DMA,
        pltpu.SemaphoreType.DMA,
    ],
)(_sc_body)


def _tc_body(sums_ref, cnt_ref, xt_ref, idt_ref, emb_ref, W_ref, b_ref, out_ref):
    sums = sums_ref[0] + sums_ref[1]
    seg_iota = jax.lax.broadcasted_iota(jnp.int32, (NUM_SEG, _TAIL), 0)
    onehot_t = (idt_ref[...] == seg_iota.astype(jnp.float32)).astype(jnp.float32)
    sums = sums + jax.lax.dot(onehot_t, xt_ref[...],
                              preferred_element_type=jnp.float32)
    counts = cnt_ref[0] + cnt_ref[1] + jnp.sum(onehot_t, axis=1)
    mean = sums / jnp.maximum(counts, 1.0)[:, None]
    cat = jnp.concatenate([mean, emb_ref[...]], axis=1)
    out_ref[...] = jax.lax.dot_general(
        cat, W_ref[...], (((1,), (1,)), ((), ())),
        preferred_element_type=jnp.float32) + b_ref[...]


def kernel(x, segment_ids, emb, W, b):
    ids = segment_ids.astype(jnp.int32)
    ids2 = jnp.zeros((_IDPAD, 1, _C), jnp.int32)
    ids2 = lax.dynamic_update_slice(
        ids2, ids[:_NCH * _C].reshape(_NCH, 1, _C), (0, 0, 0))
    ones_c = jnp.ones((_C,), jnp.float32)
    zc_c = jnp.zeros((NUM_SEG,), jnp.float32)
    sums_p, cnt_p = _sc_segsum(x, ids2, ones_c, zc_c)
    x_tail = jax.lax.slice(x, (_NCH * _C, 0), (N_NODES, HID))
    ids_tail = ids[_NCH * _C:].astype(jnp.float32).reshape(_TAIL, 1)
    out = pl.pallas_call(
        _tc_body,
        out_shape=jax.ShapeDtypeStruct((NUM_SEG, NUM_CLASS), jnp.float32),
    )(sums_p, cnt_p, x_tail, jnp.transpose(ids_tail), emb, W, b.reshape(1, NUM_CLASS))
    return out
